# baseline (speedup 1.0000x reference)
import functools

import jax
import jax.numpy as jnp
from jax import lax
from jax.experimental import pallas as pl
from jax.experimental.pallas import tpu as pltpu
from jax.experimental.pallas import tpu_sc as plsc

_B = 8
_TOTAL = 16384
_D = 4096
_CHUNK = 512
_NCHUNK = _TOTAL // _CHUNK
_NBUF = 7

_mesh = plsc.ScalarSubcoreMesh(axis_name="c")


@functools.partial(
    pl.kernel,
    out_type=jax.ShapeDtypeStruct((16,), jnp.int32),
    mesh=_mesh,
    scratch_types=[
        pltpu.SMEM((16,), jnp.int32),
        pltpu.SMEM((16,), jnp.int32),
    ],
)
def _cu_sc(len_hbm, out_hbm, lbuf, obuf):
    c = lax.axis_index("c")

    @pl.when(c == 0)
    def _():
        pltpu.sync_copy(len_hbm, lbuf)
        obuf[0] = jnp.int32(0)
        acc = jnp.int32(0)
        for i in range(_B):
            acc = acc + lbuf[i]
            obuf[i + 1] = acc
        for i in range(_B + 1, 16):
            obuf[i] = jnp.int32(0)
        pltpu.sync_copy(obuf, out_hbm)


def _copy_kernel(in_ref, out_ref, buf, rsem, wsem):
    def rd(c):
        b = c % _NBUF
        return pltpu.make_async_copy(
            in_ref.at[pl.ds(c * _CHUNK, _CHUNK), :], buf.at[b], rsem.at[b]
        )

    def wr(c):
        b = c % _NBUF
        return pltpu.make_async_copy(
            buf.at[b], out_ref.at[pl.ds(c * _CHUNK, _CHUNK), :], wsem.at[b]
        )

    for c in range(_NBUF):
        rd(c).start()
    for c in range(_NCHUNK):
        rd(c).wait()
        wr(c).start()
        nc = c + _NBUF
        if nc < _NCHUNK:
            wr(c).wait()
            rd(nc).start()
    for c in range(_NCHUNK - _NBUF, _NCHUNK):
        wr(c).wait()


def kernel(hidden_states, lengths_cpu):
    lengths16 = jnp.zeros((16,), jnp.int32).at[0:8].set(lengths_cpu.astype(jnp.int32))
    cu16 = _cu_sc(lengths16)
    cu_lengths = cu16[:9]
    values = pl.pallas_call(
        _copy_kernel,
        in_specs=[pl.BlockSpec(memory_space=pl.ANY)],
        out_specs=pl.BlockSpec(memory_space=pl.ANY),
        out_shape=jax.ShapeDtypeStruct((_TOTAL, _D), jnp.float32),
        scratch_shapes=[
            pltpu.VMEM((_NBUF, _CHUNK, _D), jnp.float32),
            pltpu.SemaphoreType.DMA((_NBUF,)),
            pltpu.SemaphoreType.DMA((_NBUF,)),
        ],
    )(hidden_states)
    return values, cu_lengths


# speedup vs baseline: 1.0909x; 1.0909x over previous
import jax
import jax.numpy as jnp
from jax.experimental import pallas as pl
from jax.experimental.pallas import tpu as pltpu

_B = 8
_TOTAL = 16384
_D = 4096
_MAXCHUNK = 512
_NBUF = 7

_SIZES = [_MAXCHUNK] * (_TOTAL // _MAXCHUNK)
_OFFS = [0]
for _s in _SIZES[:-1]:
    _OFFS.append(_OFFS[-1] + _s)
assert _OFFS[-1] + _SIZES[-1] == _TOTAL
_NCHUNK = len(_SIZES)


def _copy_cu_kernel(len_ref, in_ref, out_ref, cu_ref, buf, rsem, wsem):
    cu_ref[0] = jnp.int32(0)
    acc = jnp.int32(0)
    for i in range(_B):
        acc = acc + len_ref[i]
        cu_ref[i + 1] = acc

    def rd(c):
        b = c % _NBUF
        return pltpu.make_async_copy(
            in_ref.at[pl.ds(_OFFS[c], _SIZES[c]), :],
            buf.at[b, pl.ds(0, _SIZES[c]), :],
            rsem.at[b],
        )

    def wr(c):
        b = c % _NBUF
        return pltpu.make_async_copy(
            buf.at[b, pl.ds(0, _SIZES[c]), :],
            out_ref.at[pl.ds(_OFFS[c], _SIZES[c]), :],
            wsem.at[b],
        )

    for c in range(_NBUF):
        rd(c).start()
    for c in range(_NCHUNK):
        rd(c).wait()
        wr(c).start()
        nc = c + _NBUF
        if nc < _NCHUNK:
            wr(c).wait()
            rd(nc).start()
    for c in range(_NCHUNK - _NBUF, _NCHUNK):
        wr(c).wait()


def kernel(hidden_states, lengths_cpu):
    lengths = lengths_cpu.astype(jnp.int32)
    values, cu_lengths = pl.pallas_call(
        _copy_cu_kernel,
        in_specs=[
            pl.BlockSpec(memory_space=pltpu.SMEM),
            pl.BlockSpec(memory_space=pl.ANY),
        ],
        out_specs=[
            pl.BlockSpec(memory_space=pl.ANY),
            pl.BlockSpec(memory_space=pltpu.SMEM),
        ],
        out_shape=[
            jax.ShapeDtypeStruct((_TOTAL, _D), jnp.float32),
            jax.ShapeDtypeStruct((_B + 1,), jnp.int32),
        ],
        scratch_shapes=[
            pltpu.VMEM((_NBUF, _MAXCHUNK, _D), jnp.float32),
            pltpu.SemaphoreType.DMA((_NBUF,)),
            pltpu.SemaphoreType.DMA((_NBUF,)),
        ],
    )(lengths, hidden_states)
    return values, cu_lengths
